# baseline (speedup 1.0000x reference)
import functools

import jax
import jax.numpy as jnp
from jax import lax
from jax.experimental import pallas as pl
from jax.experimental.pallas import tpu as pltpu
from jax.experimental.pallas import tpu_sc as plsc

B = 1024
F = 128
BITS = 8
ENC = F * BITS
H = 2048
K = 32
C = 1000

NC = 2
NS = 16
NW = NC * NS
ROWS_W = H // NW
BUFW = ROWS_W * ENC

_BB = 512

_SC_MESH = plsc.VectorSubcoreMesh(
    core_axis_name="c", subcore_axis_name="s",
    num_cores=NC, num_subcores=NS)


def _sc_build_body(idx_ref, w_ref, wt_ref, buf, idx_v, w_v, *, prev, permute):
    wid = lax.axis_index("s") * NC + lax.axis_index("c")
    rows = BUFW // prev
    nchunks = ROWS_W // rows
    vec_per_row = prev // 16
    kper = K // 16

    def zero_body(i, _):
        r = i // (vec_per_row // 8)
        base = (i % (vec_per_row // 8)) * 128
        for j in range(8):
            buf[r, pl.ds(base + j * 16, 16)] = jnp.zeros((16,), jnp.float32)
        return 0

    lax.fori_loop(0, rows * (vec_per_row // 8), zero_body, 0)

    def scatter(sign):
        def body(u, _):
            r = u // kper
            iv = idx_v[r, pl.ds((u % kper) * 16, 16)]
            if permute:
                iv = (iv & (BITS - 1)) * F + (iv >> 3)
            wv = w_v[r, pl.ds((u % kper) * 16, 16)]
            rv = jnp.full((16,), r, jnp.int32)
            plsc.addupdate_scatter(buf, [rv, iv], wv * sign)
            return 0

        lax.fori_loop(0, rows * kper, body, 0)

    for cid in range(nchunks):
        row0 = wid * ROWS_W + cid * rows
        pltpu.sync_copy(idx_ref.at[pl.ds(row0, rows)], idx_v.at[pl.ds(0, rows)])
        pltpu.sync_copy(w_ref.at[pl.ds(row0, rows)], w_v.at[pl.ds(0, rows)])
        scatter(1.0)
        pltpu.sync_copy(buf, wt_ref.at[pl.ds(row0, rows)])
        if cid + 1 < nchunks:
            scatter(-1.0)


def _sc_build(idx, w, prev, permute):
    rows = BUFW // prev
    body = functools.partial(_sc_build_body, prev=prev, permute=permute)
    return pl.kernel(
        body,
        out_type=jax.ShapeDtypeStruct((H, prev), jnp.float32),
        mesh=_SC_MESH,
        scratch_types=[
            pltpu.VMEM((rows, prev), jnp.float32),
            pltpu.VMEM((ROWS_W, K), jnp.int32),
            pltpu.VMEM((ROWS_W, K), jnp.float32),
        ],
        compiler_params=pltpu.CompilerParams(needs_layout_passes=False),
    )(idx, w)


def _layer0_kernel(x_ref, wt_ref, wout_ref, act_ref, out_ref):
    x = x_ref[...]
    code = jnp.concatenate(
        [(x > (j + 0.5) * (1.0 / BITS)).astype(jnp.bfloat16)
         for j in range(BITS)], axis=1)
    wt = wt_ref[...].astype(jnp.bfloat16)
    z = lax.dot_general(code, wt, (((1,), (1,)), ((), ())),
                        preferred_element_type=jnp.float32)
    a = (z > 0.0).astype(jnp.bfloat16)
    act_ref[...] = a
    wo = wout_ref[...].astype(jnp.bfloat16)
    out_ref[...] = jnp.dot(a, wo, preferred_element_type=jnp.float32)


def _layer_kernel(act_in_ref, wt_ref, wout_ref, out_in_ref, act_ref, out_ref):
    wt = wt_ref[...].astype(jnp.bfloat16)
    z = lax.dot_general(act_in_ref[...], wt, (((1,), (1,)), ((), ())),
                        preferred_element_type=jnp.float32)
    a = (z > 0.0).astype(jnp.bfloat16)
    act_ref[...] = a
    wo = wout_ref[...].astype(jnp.bfloat16)
    out_ref[...] = out_in_ref[...] + jnp.dot(
        a, wo, preferred_element_type=jnp.float32)


def _layer0(x, wt0, wout):
    return pl.pallas_call(
        _layer0_kernel,
        grid=(B // _BB,),
        in_specs=[
            pl.BlockSpec((_BB, F), lambda i: (i, 0)),
            pl.BlockSpec((H, ENC), lambda i: (0, 0)),
            pl.BlockSpec((H, C), lambda i: (0, 0)),
        ],
        out_specs=[
            pl.BlockSpec((_BB, H), lambda i: (i, 0)),
            pl.BlockSpec((_BB, C), lambda i: (i, 0)),
        ],
        out_shape=[
            jax.ShapeDtypeStruct((B, H), jnp.bfloat16),
            jax.ShapeDtypeStruct((B, C), jnp.float32),
        ],
        compiler_params=pltpu.CompilerParams(
            dimension_semantics=("parallel",)),
    )(x, wt0, wout)


def _layer(act, wt, wout, out_in):
    return pl.pallas_call(
        _layer_kernel,
        grid=(B // _BB,),
        in_specs=[
            pl.BlockSpec((_BB, H), lambda i: (i, 0)),
            pl.BlockSpec((H, H), lambda i: (0, 0)),
            pl.BlockSpec((H, C), lambda i: (0, 0)),
            pl.BlockSpec((_BB, C), lambda i: (i, 0)),
        ],
        out_specs=[
            pl.BlockSpec((_BB, H), lambda i: (i, 0)),
            pl.BlockSpec((_BB, C), lambda i: (i, 0)),
        ],
        out_shape=[
            jax.ShapeDtypeStruct((B, H), jnp.bfloat16),
            jax.ShapeDtypeStruct((B, C), jnp.float32),
        ],
        compiler_params=pltpu.CompilerParams(
            dimension_semantics=("parallel",)),
    )(act, wt, wout, out_in)


def kernel(x, idx0, w0, idx1, w1, idx2, w2, Wout):
    wt0 = _sc_build(idx0, w0, ENC, True)
    wt1 = _sc_build(idx1, w1, H, False)
    wt2 = _sc_build(idx2, w2, H, False)
    act1, out0 = _layer0(x, wt0, Wout[0])
    act2, out1 = _layer(act1, wt1, Wout[1], out0)
    _, out2 = _layer(act2, wt2, Wout[2], out1)
    return out2


# speedup vs baseline: 1.0903x; 1.0903x over previous
import functools

import jax
import jax.numpy as jnp
from jax import lax
from jax.experimental import pallas as pl
from jax.experimental.pallas import tpu as pltpu
from jax.experimental.pallas import tpu_sc as plsc

B = 1024
F = 128
BITS = 8
ENC = F * BITS
H = 2048
K = 32
C = 1000

NC = 2
NS = 16
NW = NC * NS
ROWS_W = H // NW
BUFW = ROWS_W * ENC

_HB = 512
_OB = 512

_SC_MESH = plsc.VectorSubcoreMesh(
    core_axis_name="c", subcore_axis_name="s",
    num_cores=NC, num_subcores=NS)


def _sc_build_body(idx_ref, w_ref, wt_ref, buf, idx_v, w_v, *, prev, permute):
    wid = lax.axis_index("s") * NC + lax.axis_index("c")
    rows = BUFW // prev
    nchunks = ROWS_W // rows
    vec_per_row = prev // 16
    kper = K // 16

    def zero_body(i, _):
        r = i // (vec_per_row // 8)
        base = (i % (vec_per_row // 8)) * 128
        for j in range(8):
            buf[r, pl.ds(base + j * 16, 16)] = jnp.zeros((16,), jnp.float32)
        return 0

    lax.fori_loop(0, rows * (vec_per_row // 8), zero_body, 0)

    def scatter(sign):
        def body(u, _):
            r = u // kper
            iv = idx_v[r, pl.ds((u % kper) * 16, 16)]
            if permute:
                iv = (iv & (BITS - 1)) * F + (iv >> 3)
            wv = w_v[r, pl.ds((u % kper) * 16, 16)]
            rv = jnp.full((16,), r, jnp.int32)
            plsc.addupdate_scatter(buf, [rv, iv], wv * sign)
            return 0

        lax.fori_loop(0, rows * kper, body, 0)

    for cid in range(nchunks):
        row0 = wid * ROWS_W + cid * rows
        pltpu.sync_copy(idx_ref.at[pl.ds(row0, rows)], idx_v.at[pl.ds(0, rows)])
        pltpu.sync_copy(w_ref.at[pl.ds(row0, rows)], w_v.at[pl.ds(0, rows)])
        scatter(1.0)
        pltpu.sync_copy(buf, wt_ref.at[pl.ds(row0, rows)])
        if cid + 1 < nchunks:
            scatter(-1.0)


def _sc_build(idx, w, prev, permute):
    rows = BUFW // prev
    body = functools.partial(_sc_build_body, prev=prev, permute=permute)
    return pl.kernel(
        body,
        out_type=jax.ShapeDtypeStruct((H, prev), jnp.float32),
        mesh=_SC_MESH,
        scratch_types=[
            pltpu.VMEM((rows, prev), jnp.float32),
            pltpu.VMEM((ROWS_W, K), jnp.int32),
            pltpu.VMEM((ROWS_W, K), jnp.float32),
        ],
        compiler_params=pltpu.CompilerParams(needs_layout_passes=False),
    )(idx, w)


def _wout_cast_kernel(src_ref, dst_ref):
    dst_ref[...] = src_ref[...].astype(jnp.bfloat16)


def _wout_cast(wout):
    return pl.pallas_call(
        _wout_cast_kernel,
        grid=(3, H // 512),
        in_specs=[pl.BlockSpec((1, 512, C), lambda l, i: (l, i, 0))],
        out_specs=pl.BlockSpec((1, 512, C), lambda l, i: (l, i, 0)),
        out_shape=jax.ShapeDtypeStruct((3, H, C), jnp.bfloat16),
        compiler_params=pltpu.CompilerParams(
            dimension_semantics=("parallel", "parallel")),
    )(wout)


def _act0_kernel(x_ref, wt_ref, act_ref):
    x = x_ref[...]
    code = jnp.concatenate(
        [(x > (j + 0.5) * (1.0 / BITS)).astype(jnp.bfloat16)
         for j in range(BITS)], axis=1)
    wt = wt_ref[...].astype(jnp.bfloat16)
    z = lax.dot_general(code, wt, (((1,), (1,)), ((), ())),
                        preferred_element_type=jnp.float32)
    act_ref[...] = (z > 0.0).astype(jnp.bfloat16)


def _act_kernel(act_in_ref, wt_ref, act_ref):
    wt = wt_ref[...].astype(jnp.bfloat16)
    z = lax.dot_general(act_in_ref[...], wt, (((1,), (1,)), ((), ())),
                        preferred_element_type=jnp.float32)
    act_ref[...] = (z > 0.0).astype(jnp.bfloat16)


def _act0(x, wt0):
    return pl.pallas_call(
        _act0_kernel,
        grid=(H // _HB,),
        in_specs=[
            pl.BlockSpec((B, F), lambda i: (0, 0)),
            pl.BlockSpec((_HB, ENC), lambda i: (i, 0)),
        ],
        out_specs=pl.BlockSpec((B, _HB), lambda i: (0, i)),
        out_shape=jax.ShapeDtypeStruct((B, H), jnp.bfloat16),
        compiler_params=pltpu.CompilerParams(
            dimension_semantics=("parallel",)),
    )(x, wt0)


def _act(act_in, wt):
    return pl.pallas_call(
        _act_kernel,
        grid=(H // _HB,),
        in_specs=[
            pl.BlockSpec((B, H), lambda i: (0, 0)),
            pl.BlockSpec((_HB, H), lambda i: (i, 0)),
        ],
        out_specs=pl.BlockSpec((B, _HB), lambda i: (0, i)),
        out_shape=jax.ShapeDtypeStruct((B, H), jnp.bfloat16),
        compiler_params=pltpu.CompilerParams(
            dimension_semantics=("parallel",)),
    )(act_in, wt)


def _out_kernel(a1_ref, a2_ref, a3_ref, wo_ref, out_ref):
    acc = jnp.dot(a1_ref[...], wo_ref[0],
                  preferred_element_type=jnp.float32)
    acc = acc + jnp.dot(a2_ref[...], wo_ref[1],
                        preferred_element_type=jnp.float32)
    acc = acc + jnp.dot(a3_ref[...], wo_ref[2],
                        preferred_element_type=jnp.float32)
    out_ref[...] = acc


def _out(a1, a2, a3, wo):
    return pl.pallas_call(
        _out_kernel,
        grid=(B // _OB,),
        in_specs=[
            pl.BlockSpec((_OB, H), lambda i: (i, 0)),
            pl.BlockSpec((_OB, H), lambda i: (i, 0)),
            pl.BlockSpec((_OB, H), lambda i: (i, 0)),
            pl.BlockSpec((3, H, C), lambda i: (0, 0, 0)),
        ],
        out_specs=pl.BlockSpec((_OB, C), lambda i: (i, 0)),
        out_shape=jax.ShapeDtypeStruct((B, C), jnp.float32),
        compiler_params=pltpu.CompilerParams(
            dimension_semantics=("parallel",)),
    )(a1, a2, a3, wo)


def kernel(x, idx0, w0, idx1, w1, idx2, w2, Wout):
    wo = _wout_cast(Wout)
    wt0 = _sc_build(idx0, w0, ENC, True)
    wt1 = _sc_build(idx1, w1, H, False)
    wt2 = _sc_build(idx2, w2, H, False)
    act1 = _act0(x, wt0)
    act2 = _act(act1, wt1)
    act3 = _act(act2, wt2)
    return _out(act1, act2, act3, wo)
